# Initial kernel scaffold; baseline (speedup 1.0000x reference)
import jax
import jax.numpy as jnp
from jax.experimental import pallas as pl


def kernel(x):
    raise NotImplementedError("write your pallas kernel here")


# speedup vs baseline: 3.2375x; 3.2375x over previous
import functools

import jax
import jax.numpy as jnp
from jax.experimental import pallas as pl
from jax.experimental.pallas import tpu as pltpu

_K = 20
_RB = 256


def _gf_kernel(x_ref, out_ref):
    nb = pl.program_id(1)
    n0 = nb * _RB
    _, d, N = x_ref.shape

    xb = x_ref[0]
    x8 = x_ref[0, 0:8, :]
    xr8 = x_ref[0, 0:8, pl.ds(n0, _RB)]
    row = jax.lax.broadcasted_iota(jnp.int32, (8, N), 0)
    rowr = jax.lax.broadcasted_iota(jnp.int32, (8, _RB), 0)

    x3 = jnp.where(row < 3, x8, 0.0)
    xr3 = jnp.where(rowr < 3, 2.0 * xr8, 0.0)

    inner2 = jax.lax.dot_general(x3, xr3, (((0,), (0,)), ((), ())),
                                 preferred_element_type=jnp.float32)
    ones_col = jnp.ones((8, 1), jnp.float32)
    xxcol = jax.lax.dot_general(x3 * x3, ones_col, (((0,), (0,)), ((), ())),
                                precision=jax.lax.Precision.HIGHEST,
                                preferred_element_type=jnp.float32)
    p = inner2 - xxcol

    sub = jax.lax.broadcasted_iota(jnp.int32, (N, _RB), 0)
    xr = x_ref[0, :, pl.ds(n0, _RB)]
    neg = jnp.float32(-jnp.inf)
    for t in range(_K):
        m = jnp.max(p, axis=0, keepdims=True)
        it = jnp.min(jnp.where(p == m, sub, N), axis=0, keepdims=True)
        onehot = (sub == it).astype(jnp.float32)
        feat = jax.lax.dot_general(xb, onehot, (((1,), (0,)), ((), ())),
                                   precision=jax.lax.Precision.HIGHEST,
                                   preferred_element_type=jnp.float32)
        out_ref[0, 0:d, t, :] = feat - xr
        out_ref[0, d:2 * d, t, :] = xr
        p = jnp.where(sub == it, neg, p)


def kernel(x):
    B, d, N = x.shape
    grid = (B, N // _RB)
    out = pl.pallas_call(
        _gf_kernel,
        grid=grid,
        in_specs=[pl.BlockSpec((1, d, N), lambda b, nb: (b, 0, 0))],
        out_specs=pl.BlockSpec((1, 2 * d, _K, _RB),
                               lambda b, nb: (b, 0, 0, nb)),
        out_shape=jax.ShapeDtypeStruct((B, 2 * d, _K, N), jnp.float32),
    )(x)
    return jnp.transpose(out, (0, 1, 3, 2))


# speedup vs baseline: 6.8834x; 2.1262x over previous
import functools

import jax
import jax.numpy as jnp
from jax.experimental import pallas as pl
from jax.experimental.pallas import tpu as pltpu

_K = 20
_RB = 256


def _gf_kernel(x_ref, out_ref):
    nb = pl.program_id(1)
    n0 = nb * _RB
    _, d, N = x_ref.shape

    xb = x_ref[0]
    x8 = x_ref[0, 0:8, :]
    xr8 = x_ref[0, 0:8, pl.ds(n0, _RB)]
    row = jax.lax.broadcasted_iota(jnp.int32, (8, N), 0)
    rowr = jax.lax.broadcasted_iota(jnp.int32, (8, _RB), 0)

    x3 = jnp.where(row < 3, x8, 0.0)
    xr3 = jnp.where(rowr < 3, 2.0 * xr8, 0.0)

    inner2 = jax.lax.dot_general(x3, xr3, (((0,), (0,)), ((), ())),
                                 preferred_element_type=jnp.float32)
    ones_col = jnp.ones((8, 1), jnp.float32)
    xxcol = jax.lax.dot_general(x3 * x3, ones_col, (((0,), (0,)), ((), ())),
                                precision=jax.lax.Precision.HIGHEST,
                                preferred_element_type=jnp.float32)
    p = inner2 - xxcol

    sub = jax.lax.broadcasted_iota(jnp.int32, (N, _RB), 0)
    xr = x_ref[0, :, pl.ds(n0, _RB)]
    xb_bf = xb.astype(jnp.bfloat16)
    neg = jnp.float32(-jnp.inf)
    for t in range(_K):
        m = jnp.max(p, axis=0, keepdims=True)
        it = jnp.min(jnp.where(p == m, sub, N), axis=0, keepdims=True)
        sel = sub == it
        onehot = sel.astype(jnp.bfloat16)
        feat = jax.lax.dot_general(xb_bf, onehot, (((1,), (0,)), ((), ())),
                                   preferred_element_type=jnp.float32)
        out_ref[0, 0:d, t, :] = feat - xr
        out_ref[0, d:2 * d, t, :] = xr
        p = jnp.where(sel, neg, p)


def kernel(x):
    B, d, N = x.shape
    grid = (B, N // _RB)
    out = pl.pallas_call(
        _gf_kernel,
        grid=grid,
        in_specs=[pl.BlockSpec((1, d, N), lambda b, nb: (b, 0, 0))],
        out_specs=pl.BlockSpec((1, 2 * d, _K, _RB),
                               lambda b, nb: (b, 0, 0, nb)),
        out_shape=jax.ShapeDtypeStruct((B, 2 * d, _K, N), jnp.float32),
    )(x)
    return jnp.transpose(out, (0, 1, 3, 2))
